# baseline (device time: 100241 ns/iter reference)
import jax
import jax.numpy as jnp
from jax import lax
from jax.experimental import pallas as pl
from jax.experimental.pallas import tpu as pltpu

N_DEV = 4
M = 4096
D = 1024
M_BLK = M // N_DEV
D_HALF = D // 2

BF = jnp.bfloat16
F32 = jnp.float32


def _rows(c):
    return pl.ds(c * M_BLK, M_BLK)


def _body(x_ref, w1_ref, w2_ref, out_ref, h_buf, rs_r, rs_l,
          sems_sr, sems_rr, sems_sl, sems_rl,
          ag_sr, ag_rr, ag_sl, ag_rl):
    my = lax.axis_index("i")
    left = (my - 1) % N_DEV
    right = (my + 1) % N_DEV
    CR = pl.ds(0, D_HALF)
    CL = pl.ds(D_HALF, D_HALF)

    barrier_sem = pltpu.get_barrier_semaphore()
    for nbr in (left, right):
        pl.semaphore_signal(
            barrier_sem, inc=1,
            device_id=(nbr,), device_id_type=pl.DeviceIdType.MESH,
        )
    pl.semaphore_wait(barrier_sem, 2)

    def gemm1(c):
        h_buf[_rows(c), :] = jnp.dot(
            x_ref[_rows(c), :], w1_ref[:, :],
            preferred_element_type=F32,
        ).astype(BF)

    def gemm1_half(c, cols):
        h_buf[_rows(c), cols] = jnp.dot(
            x_ref[_rows(c), :], w1_ref[:, cols],
            preferred_element_type=F32,
        ).astype(BF)

    def rs_send_r(s, src_r):
        r = pltpu.make_async_remote_copy(
            src_ref=src_r, dst_ref=rs_r.at[s],
            send_sem=sems_sr.at[s], recv_sem=sems_rr.at[s],
            device_id=(right,), device_id_type=pl.DeviceIdType.MESH,
        )
        r.start()
        return r

    def rs_send_l(s, src_l):
        l = pltpu.make_async_remote_copy(
            src_ref=src_l, dst_ref=rs_l.at[s],
            send_sem=sems_sl.at[s], recv_sem=sems_rl.at[s],
            device_id=(left,), device_id_type=pl.DeviceIdType.MESH,
        )
        l.start()
        return l

    def rs_send(s, src_r, src_l):
        return rs_send_r(s, src_r), rs_send_l(s, src_l)

    gemm1_half(my % N_DEV, CR)
    s0_r = rs_send_r(0, h_buf.at[_rows(my), CR])
    gemm1_half(my % N_DEV, CL)
    s0_l = rs_send_l(0, h_buf.at[_rows(my), CL])
    gemm1((my + 1) % N_DEV)
    gemm1((my - 1) % N_DEV)
    gemm1((my + 2) % N_DEV)

    s0_r.wait()
    rs_r[0, :, :] = rs_r[0, :, :] + h_buf[_rows((my - 1) % N_DEV), CR]
    s0_l.wait()
    rs_l[0, :, :] = rs_l[0, :, :] + h_buf[_rows((my + 1) % N_DEV), CL]
    s1_r, s1_l = rs_send(1, rs_r.at[0], rs_l.at[0])

    s1_r.wait()
    rs_r[1, :, :] = rs_r[1, :, :] + h_buf[_rows((my - 2) % N_DEV), CR]
    s1_l.wait()
    rs_l[1, :, :] = rs_l[1, :, :] + h_buf[_rows((my + 2) % N_DEV), CL]
    s2_r, s2_l = rs_send(2, rs_r.at[1], rs_l.at[1])

    s2_r.wait()
    rs_r[2, :, :] = rs_r[2, :, :] + h_buf[_rows((my + 1) % N_DEV), CR]
    s2_l.wait()
    rs_l[2, :, :] = rs_l[2, :, :] + h_buf[_rows((my - 1) % N_DEV), CL]

    w2_top = w2_ref[0:D_HALF, :]
    w2_bot = w2_ref[D_HALF:D, :]

    def ag_send(t, ch_r, ch_l, src_r, src_l):
        r = pltpu.make_async_remote_copy(
            src_ref=src_r, dst_ref=h_buf.at[_rows(ch_r), CR],
            send_sem=ag_sr.at[t], recv_sem=ag_rr.at[t],
            device_id=(right,), device_id_type=pl.DeviceIdType.MESH,
        )
        l = pltpu.make_async_remote_copy(
            src_ref=src_l, dst_ref=h_buf.at[_rows(ch_l), CL],
            send_sem=ag_sl.at[t], recv_sem=ag_rl.at[t],
            device_id=(left,), device_id_type=pl.DeviceIdType.MESH,
        )
        r.start()
        l.start()
        return r, l

    t0_r, t0_l = ag_send(
        0, (my + 1) % N_DEV, (my - 1) % N_DEV, rs_r.at[2], rs_l.at[2]
    )
    out_ref[_rows((my + 1) % N_DEV), :] = jnp.dot(
        rs_r[2, :, :], w2_top, preferred_element_type=F32
    ).astype(BF)
    out_ref[_rows((my - 1) % N_DEV), :] = jnp.dot(
        rs_l[2, :, :], w2_bot, preferred_element_type=F32
    ).astype(BF)

    t0_r.wait()
    t0_l.wait()
    t1_r, t1_l = ag_send(
        1, my % N_DEV, my % N_DEV,
        h_buf.at[_rows(my), CR], h_buf.at[_rows(my), CL],
    )
    out_ref[_rows(my), :] = (
        jnp.dot(h_buf[_rows(my), CR], w2_top, preferred_element_type=F32)
        + jnp.dot(h_buf[_rows(my), CL], w2_bot, preferred_element_type=F32)
    ).astype(BF)

    t1_r.wait()
    t1_l.wait()
    t2_r, t2_l = ag_send(
        2, (my - 1) % N_DEV, (my + 1) % N_DEV,
        h_buf.at[_rows((my - 1) % N_DEV), CR],
        h_buf.at[_rows((my + 1) % N_DEV), CL],
    )
    out_ref[_rows((my - 1) % N_DEV), :] = (
        out_ref[_rows((my - 1) % N_DEV), :]
        + jnp.dot(h_buf[_rows((my - 1) % N_DEV), CR], w2_top,
                  preferred_element_type=F32)
    ).astype(BF)
    out_ref[_rows((my + 1) % N_DEV), :] = (
        out_ref[_rows((my + 1) % N_DEV), :]
        + jnp.dot(h_buf[_rows((my + 1) % N_DEV), CL], w2_bot,
                  preferred_element_type=F32)
    ).astype(BF)

    t2_r.wait()
    t2_l.wait()
    c2 = (my + 2) % N_DEV
    out_ref[_rows(c2), :] = (
        jnp.dot(h_buf[_rows(c2), CR], w2_top, preferred_element_type=F32)
        + jnp.dot(h_buf[_rows(c2), CL], w2_bot, preferred_element_type=F32)
    ).astype(BF)


def kernel(x, W1, W2):
    xb = x.astype(BF)
    W1b = W1.astype(BF)
    W2b = W2.astype(BF)

    sem3 = pltpu.SemaphoreType.DMA((N_DEV - 1,))
    return pl.pallas_call(
        _body,
        out_shape=jax.ShapeDtypeStruct((M, D), BF),
        in_specs=[
            pl.BlockSpec(memory_space=pltpu.VMEM),
            pl.BlockSpec(memory_space=pltpu.VMEM),
            pl.BlockSpec(memory_space=pltpu.VMEM),
        ],
        out_specs=pl.BlockSpec(memory_space=pltpu.VMEM),
        scratch_shapes=[
            pltpu.VMEM((M, D), BF),
            pltpu.VMEM((N_DEV - 1, M_BLK, D_HALF), BF),
            pltpu.VMEM((N_DEV - 1, M_BLK, D_HALF), BF),
            sem3, sem3, sem3, sem3,
            sem3, sem3, sem3, sem3,
        ],
        compiler_params=pltpu.CompilerParams(collective_id=0),
    )(xb, W1b, W2b)


# device time: 95884 ns/iter; 1.0454x vs baseline; 1.0454x over previous
import jax
import jax.numpy as jnp
from jax import lax
from jax.experimental import pallas as pl
from jax.experimental.pallas import tpu as pltpu

N_DEV = 4
M = 4096
D = 1024
M_BLK = M // N_DEV
D_HALF = D // 2

BF = jnp.bfloat16
F32 = jnp.float32


def _rows(c):
    return pl.ds(c * M_BLK, M_BLK)


def _body(x_ref, w1_ref, w2_ref, out_ref, h_buf, rs_r, rs_l, xa, xb,
          ld_sems, sems_sr, sems_rr, sems_sl, sems_rl,
          ag_sr, ag_rr, ag_sl, ag_rl):
    my = lax.axis_index("i")
    left = (my - 1) % N_DEV
    right = (my + 1) % N_DEV
    CR = pl.ds(0, D_HALF)
    CL = pl.ds(D_HALF, D_HALF)

    barrier_sem = pltpu.get_barrier_semaphore()
    for nbr in (left, right):
        pl.semaphore_signal(
            barrier_sem, inc=1,
            device_id=(nbr,), device_id_type=pl.DeviceIdType.MESH,
        )
    pl.semaphore_wait(barrier_sem, 2)

    def x_load(c, slot, i):
        cp = pltpu.make_async_copy(
            x_ref.at[_rows(c), :], slot, ld_sems.at[i]
        )
        cp.start()
        return cp

    def gemm1(c, slot):
        h_buf[_rows(c), :] = jnp.dot(
            slot[:, :].astype(BF), w1_ref[:, :],
            preferred_element_type=F32,
        ).astype(BF)

    def gemm1_half(c, slot, cols):
        h_buf[_rows(c), cols] = jnp.dot(
            slot[:, :].astype(BF), w1_ref[:, cols],
            preferred_element_type=F32,
        ).astype(BF)

    def rs_send_r(s, src_r):
        r = pltpu.make_async_remote_copy(
            src_ref=src_r, dst_ref=rs_r.at[s],
            send_sem=sems_sr.at[s], recv_sem=sems_rr.at[s],
            device_id=(right,), device_id_type=pl.DeviceIdType.MESH,
        )
        r.start()
        return r

    def rs_send_l(s, src_l):
        l = pltpu.make_async_remote_copy(
            src_ref=src_l, dst_ref=rs_l.at[s],
            send_sem=sems_sl.at[s], recv_sem=sems_rl.at[s],
            device_id=(left,), device_id_type=pl.DeviceIdType.MESH,
        )
        l.start()
        return l

    def rs_send(s, src_r, src_l):
        return rs_send_r(s, src_r), rs_send_l(s, src_l)

    ld0 = x_load(my % N_DEV, xa, 0)
    ld1 = x_load((my + 1) % N_DEV, xb, 1)
    ld0.wait()
    gemm1_half(my % N_DEV, xa, CR)
    s0_r = rs_send_r(0, h_buf.at[_rows(my), CR])
    gemm1_half(my % N_DEV, xa, CL)
    s0_l = rs_send_l(0, h_buf.at[_rows(my), CL])
    ld2 = x_load((my - 1) % N_DEV, xa, 2)
    ld1.wait()
    gemm1((my + 1) % N_DEV, xb)
    ld3 = x_load((my + 2) % N_DEV, xb, 3)
    ld2.wait()
    gemm1((my - 1) % N_DEV, xa)
    ld3.wait()
    gemm1((my + 2) % N_DEV, xb)

    s0_r.wait()
    rs_r[0, :, :] = rs_r[0, :, :] + h_buf[_rows((my - 1) % N_DEV), CR]
    s0_l.wait()
    rs_l[0, :, :] = rs_l[0, :, :] + h_buf[_rows((my + 1) % N_DEV), CL]
    s1_r, s1_l = rs_send(1, rs_r.at[0], rs_l.at[0])

    s1_r.wait()
    rs_r[1, :, :] = rs_r[1, :, :] + h_buf[_rows((my - 2) % N_DEV), CR]
    s1_l.wait()
    rs_l[1, :, :] = rs_l[1, :, :] + h_buf[_rows((my + 2) % N_DEV), CL]
    s2_r, s2_l = rs_send(2, rs_r.at[1], rs_l.at[1])

    s2_r.wait()
    rs_r[2, :, :] = rs_r[2, :, :] + h_buf[_rows((my + 1) % N_DEV), CR]
    s2_l.wait()
    rs_l[2, :, :] = rs_l[2, :, :] + h_buf[_rows((my - 1) % N_DEV), CL]

    w2_top = w2_ref[0:D_HALF, :]
    w2_bot = w2_ref[D_HALF:D, :]

    def ag_send(t, ch_r, ch_l, src_r, src_l):
        r = pltpu.make_async_remote_copy(
            src_ref=src_r, dst_ref=h_buf.at[_rows(ch_r), CR],
            send_sem=ag_sr.at[t], recv_sem=ag_rr.at[t],
            device_id=(right,), device_id_type=pl.DeviceIdType.MESH,
        )
        l = pltpu.make_async_remote_copy(
            src_ref=src_l, dst_ref=h_buf.at[_rows(ch_l), CL],
            send_sem=ag_sl.at[t], recv_sem=ag_rl.at[t],
            device_id=(left,), device_id_type=pl.DeviceIdType.MESH,
        )
        r.start()
        l.start()
        return r, l

    t0_r, t0_l = ag_send(
        0, (my + 1) % N_DEV, (my - 1) % N_DEV, rs_r.at[2], rs_l.at[2]
    )
    out_ref[_rows((my + 1) % N_DEV), :] = jnp.dot(
        rs_r[2, :, :], w2_top, preferred_element_type=F32
    ).astype(BF)
    out_ref[_rows((my - 1) % N_DEV), :] = jnp.dot(
        rs_l[2, :, :], w2_bot, preferred_element_type=F32
    ).astype(BF)

    t0_r.wait()
    t0_l.wait()
    t1_r, t1_l = ag_send(
        1, my % N_DEV, my % N_DEV,
        h_buf.at[_rows(my), CR], h_buf.at[_rows(my), CL],
    )
    out_ref[_rows(my), :] = (
        jnp.dot(h_buf[_rows(my), CR], w2_top, preferred_element_type=F32)
        + jnp.dot(h_buf[_rows(my), CL], w2_bot, preferred_element_type=F32)
    ).astype(BF)

    t1_r.wait()
    t1_l.wait()
    t2_r, t2_l = ag_send(
        2, (my - 1) % N_DEV, (my + 1) % N_DEV,
        h_buf.at[_rows((my - 1) % N_DEV), CR],
        h_buf.at[_rows((my + 1) % N_DEV), CL],
    )
    out_ref[_rows((my - 1) % N_DEV), :] = (
        out_ref[_rows((my - 1) % N_DEV), :]
        + jnp.dot(h_buf[_rows((my - 1) % N_DEV), CR], w2_top,
                  preferred_element_type=F32)
    ).astype(BF)
    out_ref[_rows((my + 1) % N_DEV), :] = (
        out_ref[_rows((my + 1) % N_DEV), :]
        + jnp.dot(h_buf[_rows((my + 1) % N_DEV), CL], w2_bot,
                  preferred_element_type=F32)
    ).astype(BF)

    t2_r.wait()
    t2_l.wait()
    c2 = (my + 2) % N_DEV
    out_ref[_rows(c2), :] = (
        jnp.dot(h_buf[_rows(c2), CR], w2_top, preferred_element_type=F32)
        + jnp.dot(h_buf[_rows(c2), CL], w2_bot, preferred_element_type=F32)
    ).astype(BF)


def kernel(x, W1, W2):
    W1b = W1.astype(BF)
    W2b = W2.astype(BF)

    sem3 = pltpu.SemaphoreType.DMA((N_DEV - 1,))
    return pl.pallas_call(
        _body,
        out_shape=jax.ShapeDtypeStruct((M, D), BF),
        in_specs=[
            pl.BlockSpec(memory_space=pltpu.MemorySpace.HBM),
            pl.BlockSpec(memory_space=pltpu.VMEM),
            pl.BlockSpec(memory_space=pltpu.VMEM),
        ],
        out_specs=pl.BlockSpec(memory_space=pltpu.VMEM),
        scratch_shapes=[
            pltpu.VMEM((M, D), BF),
            pltpu.VMEM((N_DEV - 1, M_BLK, D_HALF), BF),
            pltpu.VMEM((N_DEV - 1, M_BLK, D_HALF), BF),
            pltpu.VMEM((M_BLK, D), jnp.float32),
            pltpu.VMEM((M_BLK, D), jnp.float32),
            pltpu.SemaphoreType.DMA((4,)),
            sem3, sem3, sem3, sem3,
            sem3, sem3, sem3, sem3,
        ],
        compiler_params=pltpu.CompilerParams(collective_id=0),
    )(x, W1b, W2b)


# device time: 87429 ns/iter; 1.1465x vs baseline; 1.0967x over previous
import jax
import jax.numpy as jnp
from jax import lax
from jax.experimental import pallas as pl
from jax.experimental.pallas import tpu as pltpu

N_DEV = 4
M = 4096
D = 1024
M_BLK = M // N_DEV
M_PC = M_BLK // 2
D_HALF = D // 2

BF = jnp.bfloat16
F32 = jnp.float32


def _rows(c):
    return pl.ds(c * M_BLK, M_BLK)


def _rows_p(c, p):
    return pl.ds(c * M_BLK + p * M_PC, M_PC)


def _body(x_ref, w1_ref, w2_ref, out_ref, h_buf, rs_r, rs_l, xa, xb,
          ld_sems, sems_sr, sems_rr, sems_sl, sems_rl,
          ag_sr, ag_rr, ag_sl, ag_rl):
    my = lax.axis_index("i")
    left = (my - 1) % N_DEV
    right = (my + 1) % N_DEV
    CR = pl.ds(0, D_HALF)
    CL = pl.ds(D_HALF, D_HALF)
    PR = [pl.ds(0, M_PC), pl.ds(M_PC, M_PC)]

    barrier_sem = pltpu.get_barrier_semaphore()
    for nbr in (left, right):
        pl.semaphore_signal(
            barrier_sem, inc=1,
            device_id=(nbr,), device_id_type=pl.DeviceIdType.MESH,
        )
    pl.semaphore_wait(barrier_sem, 2)

    def x_load(c, slot, i):
        cp = pltpu.make_async_copy(
            x_ref.at[_rows(c), :], slot, ld_sems.at[i]
        )
        cp.start()
        return cp

    def gemm1(c, slot):
        h_buf[_rows(c), :] = jnp.dot(
            slot[:, :].astype(BF), w1_ref[:, :],
            preferred_element_type=F32,
        ).astype(BF)

    def gemm1_half(c, slot, cols):
        h_buf[_rows(c), cols] = jnp.dot(
            slot[:, :].astype(BF), w1_ref[:, cols],
            preferred_element_type=F32,
        ).astype(BF)

    def rs_send(s, p, dir_r):
        if dir_r:
            buf, ss, rs_, dev = rs_r, sems_sr, sems_rr, right
            cols = CR
        else:
            buf, ss, rs_, dev = rs_l, sems_sl, sems_rl, left
            cols = CL
        if s == 0:
            src = h_buf.at[_rows_p(my, p), cols]
        else:
            src = buf.at[s - 1, PR[p], :]
        cp = pltpu.make_async_remote_copy(
            src_ref=src, dst_ref=buf.at[s, PR[p], :],
            send_sem=ss.at[s, p], recv_sem=rs_.at[s, p],
            device_id=(dev,), device_id_type=pl.DeviceIdType.MESH,
        )
        cp.start()
        return cp

    def rs_add(s, p, dir_r):
        if dir_r:
            c = (my - s - 1) % N_DEV
            rs_r[s, PR[p], :] = rs_r[s, PR[p], :] + h_buf[_rows_p(c, p), CR]
        else:
            c = (my + s + 1) % N_DEV
            rs_l[s, PR[p], :] = rs_l[s, PR[p], :] + h_buf[_rows_p(c, p), CL]

    def ag_send(t, p, dir_r, ch, src):
        if dir_r:
            ss, rs_, dev, cols = ag_sr, ag_rr, right, CR
        else:
            ss, rs_, dev, cols = ag_sl, ag_rl, left, CL
        cp = pltpu.make_async_remote_copy(
            src_ref=src, dst_ref=h_buf.at[_rows_p(ch, p), cols],
            send_sem=ss.at[t, p], recv_sem=rs_.at[t, p],
            device_id=(dev,), device_id_type=pl.DeviceIdType.MESH,
        )
        cp.start()
        return cp

    ld0 = x_load(my % N_DEV, xa, 0)
    ld1 = x_load((my + 1) % N_DEV, xb, 1)
    ld0.wait()
    gemm1_half(my % N_DEV, xa, CR)
    s_r = {(0, 0): rs_send(0, 0, True), (0, 1): rs_send(0, 1, True)}
    gemm1_half(my % N_DEV, xa, CL)
    s_l = {(0, 0): rs_send(0, 0, False), (0, 1): rs_send(0, 1, False)}
    ld2 = x_load((my - 1) % N_DEV, xa, 2)
    ld1.wait()
    gemm1((my + 1) % N_DEV, xb)
    ld3 = x_load((my + 2) % N_DEV, xb, 3)
    ld2.wait()
    gemm1((my - 1) % N_DEV, xa)
    ld3.wait()
    gemm1((my + 2) % N_DEV, xb)

    for s in range(N_DEV - 2):
        for p in (0, 1):
            s_r[(s, p)].wait()
            rs_add(s, p, True)
            s_r[(s + 1, p)] = rs_send(s + 1, p, True)
            s_l[(s, p)].wait()
            rs_add(s, p, False)
            s_l[(s + 1, p)] = rs_send(s + 1, p, False)

    a_r = {}
    a_l = {}
    for p in (0, 1):
        s_r[(2, p)].wait()
        rs_add(2, p, True)
        a_r[(0, p)] = ag_send(0, p, True, (my + 1) % N_DEV,
                              rs_r.at[2, PR[p], :])
        s_l[(2, p)].wait()
        rs_add(2, p, False)
        a_l[(0, p)] = ag_send(0, p, False, (my - 1) % N_DEV,
                              rs_l.at[2, PR[p], :])

    w2_top = w2_ref[0:D_HALF, :]
    w2_bot = w2_ref[D_HALF:D, :]
    out_ref[_rows((my + 1) % N_DEV), :] = jnp.dot(
        rs_r[2, :, :], w2_top, preferred_element_type=F32
    ).astype(BF)
    out_ref[_rows((my - 1) % N_DEV), :] = jnp.dot(
        rs_l[2, :, :], w2_bot, preferred_element_type=F32
    ).astype(BF)

    for p in (0, 1):
        a_r[(0, p)].wait()
        a_l[(0, p)].wait()
        a_r[(1, p)] = ag_send(1, p, True, my % N_DEV,
                              h_buf.at[_rows_p(my, p), CR])
        a_l[(1, p)] = ag_send(1, p, False, my % N_DEV,
                              h_buf.at[_rows_p(my, p), CL])
    out_ref[_rows(my), :] = (
        jnp.dot(h_buf[_rows(my), CR], w2_top, preferred_element_type=F32)
        + jnp.dot(h_buf[_rows(my), CL], w2_bot, preferred_element_type=F32)
    ).astype(BF)

    for p in (0, 1):
        a_r[(1, p)].wait()
        a_l[(1, p)].wait()
        a_r[(2, p)] = ag_send(2, p, True, (my - 1) % N_DEV,
                              h_buf.at[_rows_p((my - 1) % N_DEV, p), CR])
        a_l[(2, p)] = ag_send(2, p, False, (my + 1) % N_DEV,
                              h_buf.at[_rows_p((my + 1) % N_DEV, p), CL])
    out_ref[_rows((my - 1) % N_DEV), :] = (
        out_ref[_rows((my - 1) % N_DEV), :]
        + jnp.dot(h_buf[_rows((my - 1) % N_DEV), CR], w2_top,
                  preferred_element_type=F32)
    ).astype(BF)
    out_ref[_rows((my + 1) % N_DEV), :] = (
        out_ref[_rows((my + 1) % N_DEV), :]
        + jnp.dot(h_buf[_rows((my + 1) % N_DEV), CL], w2_bot,
                  preferred_element_type=F32)
    ).astype(BF)

    for p in (0, 1):
        a_r[(2, p)].wait()
        a_l[(2, p)].wait()
    c2 = (my + 2) % N_DEV
    out_ref[_rows(c2), :] = (
        jnp.dot(h_buf[_rows(c2), CR], w2_top, preferred_element_type=F32)
        + jnp.dot(h_buf[_rows(c2), CL], w2_bot, preferred_element_type=F32)
    ).astype(BF)


def kernel(x, W1, W2):
    W1b = W1.astype(BF)
    W2b = W2.astype(BF)

    sem32 = pltpu.SemaphoreType.DMA((N_DEV - 1, 2))
    return pl.pallas_call(
        _body,
        out_shape=jax.ShapeDtypeStruct((M, D), BF),
        in_specs=[
            pl.BlockSpec(memory_space=pltpu.MemorySpace.HBM),
            pl.BlockSpec(memory_space=pltpu.VMEM),
            pl.BlockSpec(memory_space=pltpu.VMEM),
        ],
        out_specs=pl.BlockSpec(memory_space=pltpu.VMEM),
        scratch_shapes=[
            pltpu.VMEM((M, D), BF),
            pltpu.VMEM((N_DEV - 1, M_BLK, D_HALF), BF),
            pltpu.VMEM((N_DEV - 1, M_BLK, D_HALF), BF),
            pltpu.VMEM((M_BLK, D), jnp.float32),
            pltpu.VMEM((M_BLK, D), jnp.float32),
            pltpu.SemaphoreType.DMA((4,)),
            sem32, sem32, sem32, sem32,
            sem32, sem32, sem32, sem32,
        ],
        compiler_params=pltpu.CompilerParams(collective_id=0),
    )(x, W1b, W2b)


# device time: 86583 ns/iter; 1.1577x vs baseline; 1.0098x over previous
import jax
import jax.numpy as jnp
from jax import lax
from jax.experimental import pallas as pl
from jax.experimental.pallas import tpu as pltpu

N_DEV = 4
M = 4096
D = 1024
M_BLK = M // N_DEV
NP = 4
M_PC = M_BLK // NP
D_HALF = D // 2

BF = jnp.bfloat16
F32 = jnp.float32


def _rows(c):
    return pl.ds(c * M_BLK, M_BLK)


def _rows_p(c, p):
    return pl.ds(c * M_BLK + p * M_PC, M_PC)


def _body(x_ref, w1_ref, w2_ref, out_ref, h_buf, rs_r, rs_l, xa, xb,
          ld_sems, sems_sr, sems_rr, sems_sl, sems_rl,
          ag_sr, ag_rr, ag_sl, ag_rl):
    my = lax.axis_index("i")
    left = (my - 1) % N_DEV
    right = (my + 1) % N_DEV
    CR = pl.ds(0, D_HALF)
    CL = pl.ds(D_HALF, D_HALF)
    PR = [pl.ds(i * M_PC, M_PC) for i in range(NP)]

    barrier_sem = pltpu.get_barrier_semaphore()
    for nbr in (left, right):
        pl.semaphore_signal(
            barrier_sem, inc=1,
            device_id=(nbr,), device_id_type=pl.DeviceIdType.MESH,
        )

    def x_load(c, slot, i):
        cp = pltpu.make_async_copy(
            x_ref.at[_rows(c), :], slot, ld_sems.at[i]
        )
        cp.start()
        return cp

    def gemm1(c, slot):
        h_buf[_rows(c), :] = jnp.dot(
            slot[:, :].astype(BF), w1_ref[:, :],
            preferred_element_type=F32,
        ).astype(BF)

    def gemm1_half(c, slot, cols):
        h_buf[_rows(c), cols] = jnp.dot(
            slot[:, :].astype(BF), w1_ref[:, cols],
            preferred_element_type=F32,
        ).astype(BF)

    def rs_send(s, p, dir_r):
        if dir_r:
            buf, ss, rs_, dev = rs_r, sems_sr, sems_rr, right
            cols = CR
        else:
            buf, ss, rs_, dev = rs_l, sems_sl, sems_rl, left
            cols = CL
        if s == 0:
            src = h_buf.at[_rows_p(my, p), cols]
        else:
            src = buf.at[s - 1, PR[p], :]
        cp = pltpu.make_async_remote_copy(
            src_ref=src, dst_ref=buf.at[s, PR[p], :],
            send_sem=ss.at[s, p], recv_sem=rs_.at[s, p],
            device_id=(dev,), device_id_type=pl.DeviceIdType.MESH,
        )
        cp.start()
        return cp

    def rs_add(s, p, dir_r):
        if dir_r:
            c = (my - s - 1) % N_DEV
            rs_r[s, PR[p], :] = rs_r[s, PR[p], :] + h_buf[_rows_p(c, p), CR]
        else:
            c = (my + s + 1) % N_DEV
            rs_l[s, PR[p], :] = rs_l[s, PR[p], :] + h_buf[_rows_p(c, p), CL]

    def ag_send(t, p, dir_r, ch, src):
        if dir_r:
            ss, rs_, dev, cols = ag_sr, ag_rr, right, CR
        else:
            ss, rs_, dev, cols = ag_sl, ag_rl, left, CL
        cp = pltpu.make_async_remote_copy(
            src_ref=src, dst_ref=h_buf.at[_rows_p(ch, p), cols],
            send_sem=ss.at[t, p], recv_sem=rs_.at[t, p],
            device_id=(dev,), device_id_type=pl.DeviceIdType.MESH,
        )
        cp.start()
        return cp

    ld0 = x_load(my % N_DEV, xa, 0)
    ld1 = x_load((my + 1) % N_DEV, xb, 1)
    ld0.wait()
    gemm1_half(my % N_DEV, xa, CR)
    pl.semaphore_wait(barrier_sem, 2)
    s_r = {(0, p): rs_send(0, p, True) for p in range(NP)}
    gemm1_half(my % N_DEV, xa, CL)
    s_l = {(0, p): rs_send(0, p, False) for p in range(NP)}
    ld2 = x_load((my - 1) % N_DEV, xa, 2)
    ld1.wait()
    gemm1((my + 1) % N_DEV, xb)
    ld3 = x_load((my + 2) % N_DEV, xb, 3)
    ld2.wait()
    gemm1((my - 1) % N_DEV, xa)
    ld3.wait()
    gemm1((my + 2) % N_DEV, xb)

    for s in range(N_DEV - 2):
        for p in range(NP):
            s_r[(s, p)].wait()
            rs_add(s, p, True)
            s_r[(s + 1, p)] = rs_send(s + 1, p, True)
            s_l[(s, p)].wait()
            rs_add(s, p, False)
            s_l[(s + 1, p)] = rs_send(s + 1, p, False)

    a_r = {}
    a_l = {}
    for p in range(NP):
        s_r[(2, p)].wait()
        rs_add(2, p, True)
        a_r[(0, p)] = ag_send(0, p, True, (my + 1) % N_DEV,
                              rs_r.at[2, PR[p], :])
        s_l[(2, p)].wait()
        rs_add(2, p, False)
        a_l[(0, p)] = ag_send(0, p, False, (my - 1) % N_DEV,
                              rs_l.at[2, PR[p], :])

    w2_top = w2_ref[0:D_HALF, :]
    w2_bot = w2_ref[D_HALF:D, :]
    out_ref[_rows((my + 1) % N_DEV), :] = jnp.dot(
        rs_r[2, :, :], w2_top, preferred_element_type=F32
    ).astype(BF)
    out_ref[_rows((my - 1) % N_DEV), :] = jnp.dot(
        rs_l[2, :, :], w2_bot, preferred_element_type=F32
    ).astype(BF)

    for p in range(NP):
        a_r[(0, p)].wait()
        a_l[(0, p)].wait()
        a_r[(1, p)] = ag_send(1, p, True, my % N_DEV,
                              h_buf.at[_rows_p(my, p), CR])
        a_l[(1, p)] = ag_send(1, p, False, my % N_DEV,
                              h_buf.at[_rows_p(my, p), CL])
    out_ref[_rows(my), :] = (
        jnp.dot(h_buf[_rows(my), CR], w2_top, preferred_element_type=F32)
        + jnp.dot(h_buf[_rows(my), CL], w2_bot, preferred_element_type=F32)
    ).astype(BF)

    for p in range(NP):
        a_r[(1, p)].wait()
        a_l[(1, p)].wait()
        a_r[(2, p)] = ag_send(2, p, True, (my - 1) % N_DEV,
                              h_buf.at[_rows_p((my - 1) % N_DEV, p), CR])
        a_l[(2, p)] = ag_send(2, p, False, (my + 1) % N_DEV,
                              h_buf.at[_rows_p((my + 1) % N_DEV, p), CL])
    out_ref[_rows((my - 1) % N_DEV), :] = (
        out_ref[_rows((my - 1) % N_DEV), :]
        + jnp.dot(h_buf[_rows((my - 1) % N_DEV), CR], w2_top,
                  preferred_element_type=F32)
    ).astype(BF)
    out_ref[_rows((my + 1) % N_DEV), :] = (
        out_ref[_rows((my + 1) % N_DEV), :]
        + jnp.dot(h_buf[_rows((my + 1) % N_DEV), CL], w2_bot,
                  preferred_element_type=F32)
    ).astype(BF)

    for p in range(NP):
        a_r[(2, p)].wait()
        a_l[(2, p)].wait()
    c2 = (my + 2) % N_DEV
    out_ref[_rows(c2), :] = (
        jnp.dot(h_buf[_rows(c2), CR], w2_top, preferred_element_type=F32)
        + jnp.dot(h_buf[_rows(c2), CL], w2_bot, preferred_element_type=F32)
    ).astype(BF)


def kernel(x, W1, W2):
    W1b = W1.astype(BF)
    W2b = W2.astype(BF)

    sem32 = pltpu.SemaphoreType.DMA((N_DEV - 1, NP))
    return pl.pallas_call(
        _body,
        out_shape=jax.ShapeDtypeStruct((M, D), BF),
        in_specs=[
            pl.BlockSpec(memory_space=pltpu.MemorySpace.HBM),
            pl.BlockSpec(memory_space=pltpu.VMEM),
            pl.BlockSpec(memory_space=pltpu.VMEM),
        ],
        out_specs=pl.BlockSpec(memory_space=pltpu.VMEM),
        scratch_shapes=[
            pltpu.VMEM((M, D), BF),
            pltpu.VMEM((N_DEV - 1, M_BLK, D_HALF), BF),
            pltpu.VMEM((N_DEV - 1, M_BLK, D_HALF), BF),
            pltpu.VMEM((M_BLK, D), jnp.float32),
            pltpu.VMEM((M_BLK, D), jnp.float32),
            pltpu.SemaphoreType.DMA((4,)),
            sem32, sem32, sem32, sem32,
            sem32, sem32, sem32, sem32,
        ],
        compiler_params=pltpu.CompilerParams(collective_id=0),
    )(x, W1b, W2b)


# device time: 84885 ns/iter; 1.1809x vs baseline; 1.0200x over previous
import jax
import jax.numpy as jnp
from jax import lax
from jax.experimental import pallas as pl
from jax.experimental.pallas import tpu as pltpu

N_DEV = 4
M = 4096
D = 1024
M_BLK = M // N_DEV
M_HF = M_BLK // 2
NP = 4
M_PC = M_BLK // NP
D_HALF = D // 2

BF = jnp.bfloat16
F32 = jnp.float32


def _rows(c):
    return pl.ds(c * M_BLK, M_BLK)


def _rows_h(c, h):
    return pl.ds(c * M_BLK + h * M_HF, M_HF)


def _rows_p(c, p):
    return pl.ds(c * M_BLK + p * M_PC, M_PC)


def _body(x_ref, w1_ref, w2_ref, out_ref, h_buf, rs_r, rs_l, xa, xb, o_stage,
          ld_sems, st_sems, sems_sr, sems_rr, sems_sl, sems_rl,
          ag_sr, ag_rr, ag_sl, ag_rl):
    my = lax.axis_index("i")
    left = (my - 1) % N_DEV
    right = (my + 1) % N_DEV
    CR = pl.ds(0, D_HALF)
    CL = pl.ds(D_HALF, D_HALF)
    PR = [pl.ds(i * M_PC, M_PC) for i in range(NP)]

    barrier_sem = pltpu.get_barrier_semaphore()
    for nbr in (left, right):
        pl.semaphore_signal(
            barrier_sem, inc=1,
            device_id=(nbr,), device_id_type=pl.DeviceIdType.MESH,
        )

    def x_load(c, h, slot, i):
        cp = pltpu.make_async_copy(
            x_ref.at[_rows_h(c, h), :], slot, ld_sems.at[i]
        )
        cp.start()
        return cp

    def gemm1_h(c, h, slot):
        h_buf[_rows_h(c, h), :] = jnp.dot(
            slot[:, :].astype(BF), w1_ref[:, :],
            preferred_element_type=F32,
        ).astype(BF)

    def out_store(slot, c):
        cp = pltpu.make_async_copy(
            o_stage.at[slot], out_ref.at[_rows(c), :], st_sems.at[slot]
        )
        cp.start()
        return cp

    def rs_send(s, p, dir_r):
        if dir_r:
            buf, ss, rs_, dev, cols = rs_r, sems_sr, sems_rr, right, CR
        else:
            buf, ss, rs_, dev, cols = rs_l, sems_sl, sems_rl, left, CL
        if s == 0:
            src = h_buf.at[_rows_p(my, p), cols]
        else:
            src = buf.at[s - 1, PR[p], :]
        cp = pltpu.make_async_remote_copy(
            src_ref=src, dst_ref=buf.at[s, PR[p], :],
            send_sem=ss.at[s, p], recv_sem=rs_.at[s, p],
            device_id=(dev,), device_id_type=pl.DeviceIdType.MESH,
        )
        cp.start()
        return cp

    def rs_add(s, p, dir_r):
        if dir_r:
            c = (my - s - 1) % N_DEV
            rs_r[s, PR[p], :] = rs_r[s, PR[p], :] + h_buf[_rows_p(c, p), CR]
        else:
            c = (my + s + 1) % N_DEV
            rs_l[s, PR[p], :] = rs_l[s, PR[p], :] + h_buf[_rows_p(c, p), CL]

    def ag_send(t, p, dir_r, ch, src):
        if dir_r:
            ss, rs_, dev, cols = ag_sr, ag_rr, right, CR
        else:
            ss, rs_, dev, cols = ag_sl, ag_rl, left, CL
        cp = pltpu.make_async_remote_copy(
            src_ref=src, dst_ref=h_buf.at[_rows_p(ch, p), cols],
            send_sem=ss.at[t, p], recv_sem=rs_.at[t, p],
            device_id=(dev,), device_id_type=pl.DeviceIdType.MESH,
        )
        cp.start()
        return cp

    ld = x_load(my % N_DEV, 0, xa, 0)
    ldn = x_load(my % N_DEV, 1, xb, 1)
    ld.wait()
    gemm1_h(my % N_DEV, 0, xa)
    pl.semaphore_wait(barrier_sem, 2)
    s_r = {(0, p): rs_send(0, p, True) for p in (0, 1)}
    s_l = {(0, p): rs_send(0, p, False) for p in (0, 1)}
    ld = x_load((my - 1) % N_DEV, 0, xa, 2)
    ldn.wait()
    gemm1_h(my % N_DEV, 1, xb)
    s_r[(0, 2)] = rs_send(0, 2, True)
    s_r[(0, 3)] = rs_send(0, 3, True)
    s_l[(0, 2)] = rs_send(0, 2, False)
    s_l[(0, 3)] = rs_send(0, 3, False)
    ldn = x_load((my + 1) % N_DEV, 0, xb, 3)
    ld.wait()
    gemm1_h((my - 1) % N_DEV, 0, xa)
    ld = x_load((my - 1) % N_DEV, 1, xa, 4)
    ldn.wait()
    gemm1_h((my + 1) % N_DEV, 0, xb)
    ldn = x_load((my + 1) % N_DEV, 1, xb, 5)
    ld.wait()
    gemm1_h((my - 1) % N_DEV, 1, xa)
    ld = x_load((my + 2) % N_DEV, 0, xa, 6)
    ldn.wait()
    gemm1_h((my + 1) % N_DEV, 1, xb)
    ldn = x_load((my + 2) % N_DEV, 1, xb, 7)
    ld.wait()
    gemm1_h((my + 2) % N_DEV, 0, xa)
    ldn.wait()
    gemm1_h((my + 2) % N_DEV, 1, xb)

    for s in range(N_DEV - 2):
        for p in range(NP):
            s_r[(s, p)].wait()
            rs_add(s, p, True)
            s_r[(s + 1, p)] = rs_send(s + 1, p, True)
            s_l[(s, p)].wait()
            rs_add(s, p, False)
            s_l[(s + 1, p)] = rs_send(s + 1, p, False)

    a_r = {}
    a_l = {}
    for p in range(NP):
        s_r[(2, p)].wait()
        rs_add(2, p, True)
        a_r[(0, p)] = ag_send(0, p, True, (my + 1) % N_DEV,
                              rs_r.at[2, PR[p], :])
        s_l[(2, p)].wait()
        rs_add(2, p, False)
        a_l[(0, p)] = ag_send(0, p, False, (my - 1) % N_DEV,
                              rs_l.at[2, PR[p], :])

    w2_top = w2_ref[0:D_HALF, :]
    w2_bot = w2_ref[D_HALF:D, :]
    o_stage[0, :, :] = jnp.dot(
        rs_r[2, :, :], w2_top, preferred_element_type=F32
    ).astype(BF)
    o_stage[1, :, :] = jnp.dot(
        rs_l[2, :, :], w2_bot, preferred_element_type=F32
    ).astype(BF)

    for p in range(NP):
        a_r[(0, p)].wait()
        a_l[(0, p)].wait()
        a_r[(1, p)] = ag_send(1, p, True, my % N_DEV,
                              h_buf.at[_rows_p(my, p), CR])
        a_l[(1, p)] = ag_send(1, p, False, my % N_DEV,
                              h_buf.at[_rows_p(my, p), CL])
    o_stage[2, :, :] = (
        jnp.dot(h_buf[_rows(my), CR], w2_top, preferred_element_type=F32)
        + jnp.dot(h_buf[_rows(my), CL], w2_bot, preferred_element_type=F32)
    ).astype(BF)
    st2 = out_store(2, my % N_DEV)

    for p in range(NP):
        a_r[(1, p)].wait()
        a_l[(1, p)].wait()
        a_r[(2, p)] = ag_send(2, p, True, (my - 1) % N_DEV,
                              h_buf.at[_rows_p((my - 1) % N_DEV, p), CR])
        a_l[(2, p)] = ag_send(2, p, False, (my + 1) % N_DEV,
                              h_buf.at[_rows_p((my + 1) % N_DEV, p), CL])
    o_stage[1, :, :] = (
        o_stage[1, :, :]
        + jnp.dot(h_buf[_rows((my - 1) % N_DEV), CR], w2_top,
                  preferred_element_type=F32)
    ).astype(BF)
    st1 = out_store(1, (my - 1) % N_DEV)
    o_stage[0, :, :] = (
        o_stage[0, :, :]
        + jnp.dot(h_buf[_rows((my + 1) % N_DEV), CL], w2_bot,
                  preferred_element_type=F32)
    ).astype(BF)
    st0 = out_store(0, (my + 1) % N_DEV)

    for p in range(NP):
        a_r[(2, p)].wait()
        a_l[(2, p)].wait()
    st2.wait()
    c2 = (my + 2) % N_DEV
    o_stage[2, :, :] = (
        jnp.dot(h_buf[_rows(c2), CR], w2_top, preferred_element_type=F32)
        + jnp.dot(h_buf[_rows(c2), CL], w2_bot, preferred_element_type=F32)
    ).astype(BF)
    st3 = out_store(2, c2)

    st1.wait()
    st0.wait()
    st3.wait()


def kernel(x, W1, W2):
    W1b = W1.astype(BF)
    W2b = W2.astype(BF)

    sem32 = pltpu.SemaphoreType.DMA((N_DEV - 1, NP))
    return pl.pallas_call(
        _body,
        out_shape=jax.ShapeDtypeStruct((M, D), BF),
        in_specs=[
            pl.BlockSpec(memory_space=pltpu.MemorySpace.HBM),
            pl.BlockSpec(memory_space=pltpu.VMEM),
            pl.BlockSpec(memory_space=pltpu.VMEM),
        ],
        out_specs=pl.BlockSpec(memory_space=pltpu.MemorySpace.HBM),
        scratch_shapes=[
            pltpu.VMEM((M, D), BF),
            pltpu.VMEM((N_DEV - 1, M_BLK, D_HALF), BF),
            pltpu.VMEM((N_DEV - 1, M_BLK, D_HALF), BF),
            pltpu.VMEM((M_HF, D), jnp.float32),
            pltpu.VMEM((M_HF, D), jnp.float32),
            pltpu.VMEM((3, M_BLK, D), BF),
            pltpu.SemaphoreType.DMA((8,)),
            pltpu.SemaphoreType.DMA((3,)),
            sem32, sem32, sem32, sem32,
            sem32, sem32, sem32, sem32,
        ],
        compiler_params=pltpu.CompilerParams(collective_id=0),
    )(x, W1b, W2b)


# device time: 82984 ns/iter; 1.2080x vs baseline; 1.0229x over previous
import jax
import jax.numpy as jnp
from jax import lax
from jax.experimental import pallas as pl
from jax.experimental.pallas import tpu as pltpu

N_DEV = 4
M = 4096
D = 1024
M_BLK = M // N_DEV
M_HF = M_BLK // 2
NP = 4
M_PC = M_BLK // NP
D_HALF = D // 2

BF = jnp.bfloat16
F32 = jnp.float32


def _rows(c):
    return pl.ds(c * M_BLK, M_BLK)


def _rows_h(c, h):
    return pl.ds(c * M_BLK + h * M_HF, M_HF)


def _rows_p(c, p):
    return pl.ds(c * M_BLK + p * M_PC, M_PC)


def _body(x_ref, w1_ref, w2_ref, out_ref, h_buf, rs_r, rs_l, xa, xb, o_stage,
          ld_sems, st_sems, sems_sr, sems_rr, sems_sl, sems_rl,
          ag_sr, ag_rr, ag_sl, ag_rl):
    my = lax.axis_index("i")
    left = (my - 1) % N_DEV
    right = (my + 1) % N_DEV
    CR = pl.ds(0, D_HALF)
    CL = pl.ds(D_HALF, D_HALF)
    PR = [pl.ds(i * M_PC, M_PC) for i in range(NP)]

    barrier_sem = pltpu.get_barrier_semaphore()
    for nbr in (left, right):
        pl.semaphore_signal(
            barrier_sem, inc=1,
            device_id=(nbr,), device_id_type=pl.DeviceIdType.MESH,
        )

    def x_load(c, h, slot, i):
        cp = pltpu.make_async_copy(
            x_ref.at[_rows_h(c, h), :], slot, ld_sems.at[i]
        )
        cp.start()
        return cp

    def gemm1_h(c, h, slot):
        h_buf[_rows_h(c, h), :] = jnp.dot(
            slot[:, :].astype(BF), w1_ref[:, :],
            preferred_element_type=F32,
        ).astype(BF)

    def out_store(slot, c):
        cp = pltpu.make_async_copy(
            o_stage.at[slot], out_ref.at[_rows(c), :], st_sems.at[slot]
        )
        cp.start()
        return cp

    def rs_send(s, p, dir_r):
        if dir_r:
            buf, ss, rs_, dev, cols = rs_r, sems_sr, sems_rr, right, CR
        else:
            buf, ss, rs_, dev, cols = rs_l, sems_sl, sems_rl, left, CL
        if s == 0:
            src = h_buf.at[_rows_p(my, p), cols]
        else:
            src = buf.at[s - 1, PR[p], :]
        cp = pltpu.make_async_remote_copy(
            src_ref=src, dst_ref=buf.at[s, PR[p], :],
            send_sem=ss.at[s, p], recv_sem=rs_.at[s, p],
            device_id=(dev,), device_id_type=pl.DeviceIdType.MESH,
        )
        cp.start()
        return cp

    def rs_add(s, p, dir_r):
        if dir_r:
            c = (my - s - 1) % N_DEV
            rs_r[s, PR[p], :] = rs_r[s, PR[p], :] + h_buf[_rows_p(c, p), CR]
        else:
            c = (my + s + 1) % N_DEV
            rs_l[s, PR[p], :] = rs_l[s, PR[p], :] + h_buf[_rows_p(c, p), CL]

    def ag_send(t, p, dir_r, ch, src):
        if dir_r:
            ss, rs_, dev, cols = ag_sr, ag_rr, right, CR
        else:
            ss, rs_, dev, cols = ag_sl, ag_rl, left, CL
        cp = pltpu.make_async_remote_copy(
            src_ref=src, dst_ref=h_buf.at[_rows_p(ch, p), cols],
            send_sem=ss.at[t, p], recv_sem=rs_.at[t, p],
            device_id=(dev,), device_id_type=pl.DeviceIdType.MESH,
        )
        cp.start()
        return cp

    ld = x_load(my % N_DEV, 0, xa, 0)
    ldn = x_load(my % N_DEV, 1, xb, 1)
    ld.wait()
    gemm1_h(my % N_DEV, 0, xa)
    pl.semaphore_wait(barrier_sem, 2)
    s_r = {(0, p): rs_send(0, p, True) for p in (0, 1)}
    s_l = {(0, p): rs_send(0, p, False) for p in (0, 1)}
    ld = x_load((my - 1) % N_DEV, 0, xa, 2)
    ldn.wait()
    gemm1_h(my % N_DEV, 1, xb)
    s_r[(0, 2)] = rs_send(0, 2, True)
    s_r[(0, 3)] = rs_send(0, 3, True)
    s_l[(0, 2)] = rs_send(0, 2, False)
    s_l[(0, 3)] = rs_send(0, 3, False)
    ldn = x_load((my + 1) % N_DEV, 0, xb, 3)
    ld.wait()
    gemm1_h((my - 1) % N_DEV, 0, xa)
    ld = x_load((my - 1) % N_DEV, 1, xa, 4)
    ldn.wait()
    gemm1_h((my + 1) % N_DEV, 0, xb)
    ldn = x_load((my + 1) % N_DEV, 1, xb, 5)
    ld.wait()
    gemm1_h((my - 1) % N_DEV, 1, xa)
    ld = x_load((my + 2) % N_DEV, 0, xa, 6)
    ldn.wait()
    gemm1_h((my + 1) % N_DEV, 1, xb)
    ldn = x_load((my + 2) % N_DEV, 1, xb, 7)
    ld.wait()
    gemm1_h((my + 2) % N_DEV, 0, xa)
    ldn.wait()
    gemm1_h((my + 2) % N_DEV, 1, xb)

    for s in range(N_DEV - 2):
        for p in range(NP):
            s_r[(s, p)].wait()
            rs_add(s, p, True)
            s_r[(s + 1, p)] = rs_send(s + 1, p, True)
            s_l[(s, p)].wait()
            rs_add(s, p, False)
            s_l[(s + 1, p)] = rs_send(s + 1, p, False)

    a_r = {}
    a_l = {}
    for p in range(NP):
        s_r[(2, p)].wait()
        rs_add(2, p, True)
        a_r[(0, p)] = ag_send(0, p, True, (my + 1) % N_DEV,
                              rs_r.at[2, PR[p], :])
        s_l[(2, p)].wait()
        rs_add(2, p, False)
        a_l[(0, p)] = ag_send(0, p, False, (my - 1) % N_DEV,
                              rs_l.at[2, PR[p], :])

    w2_top = w2_ref[0:D_HALF, :]
    w2_bot = w2_ref[D_HALF:D, :]
    o_stage[0, :, :] = jnp.dot(
        rs_r[2, :, :], w2_top, preferred_element_type=F32
    ).astype(BF)
    o_stage[1, :, :] = jnp.dot(
        rs_l[2, :, :], w2_bot, preferred_element_type=F32
    ).astype(BF)

    for p in range(NP):
        a_r[(0, p)].wait()
        a_l[(0, p)].wait()
        a_r[(1, p)] = ag_send(1, p, True, my % N_DEV,
                              h_buf.at[_rows_p(my, p), CR])
        a_l[(1, p)] = ag_send(1, p, False, my % N_DEV,
                              h_buf.at[_rows_p(my, p), CL])
    o_stage[2, :, :] = (
        jnp.dot(h_buf[_rows(my), CR], w2_top, preferred_element_type=F32)
        + jnp.dot(h_buf[_rows(my), CL], w2_bot, preferred_element_type=F32)
    ).astype(BF)
    st2 = out_store(2, my % N_DEV)

    for p in range(NP):
        a_r[(1, p)].wait()
        a_l[(1, p)].wait()
        a_r[(2, p)] = ag_send(2, p, True, (my - 1) % N_DEV,
                              h_buf.at[_rows_p((my - 1) % N_DEV, p), CR])
        a_l[(2, p)] = ag_send(2, p, False, (my + 1) % N_DEV,
                              h_buf.at[_rows_p((my + 1) % N_DEV, p), CL])
    o_stage[1, :, :] = (
        o_stage[1, :, :]
        + jnp.dot(h_buf[_rows((my - 1) % N_DEV), CR], w2_top,
                  preferred_element_type=F32)
    ).astype(BF)
    st1 = out_store(1, (my - 1) % N_DEV)
    o_stage[0, :, :] = (
        o_stage[0, :, :]
        + jnp.dot(h_buf[_rows((my + 1) % N_DEV), CL], w2_bot,
                  preferred_element_type=F32)
    ).astype(BF)
    st0 = out_store(0, (my + 1) % N_DEV)

    st2.wait()
    c2 = (my + 2) % N_DEV
    pc_sts = []
    for p in range(NP):
        a_r[(2, p)].wait()
        a_l[(2, p)].wait()
        o_stage[2, PR[p], :] = (
            jnp.dot(h_buf[_rows_p(c2, p), CR], w2_top,
                    preferred_element_type=F32)
            + jnp.dot(h_buf[_rows_p(c2, p), CL], w2_bot,
                      preferred_element_type=F32)
        ).astype(BF)
        cp = pltpu.make_async_copy(
            o_stage.at[2, PR[p], :], out_ref.at[_rows_p(c2, p), :],
            st_sems.at[3 + p],
        )
        cp.start()
        pc_sts.append(cp)

    st1.wait()
    st0.wait()
    for cp in pc_sts:
        cp.wait()


def kernel(x, W1, W2):
    W1b = W1.astype(BF)
    W2b = W2.astype(BF)

    sem32 = pltpu.SemaphoreType.DMA((N_DEV - 1, NP))
    return pl.pallas_call(
        _body,
        out_shape=jax.ShapeDtypeStruct((M, D), BF),
        in_specs=[
            pl.BlockSpec(memory_space=pltpu.MemorySpace.HBM),
            pl.BlockSpec(memory_space=pltpu.VMEM),
            pl.BlockSpec(memory_space=pltpu.VMEM),
        ],
        out_specs=pl.BlockSpec(memory_space=pltpu.MemorySpace.HBM),
        scratch_shapes=[
            pltpu.VMEM((M, D), BF),
            pltpu.VMEM((N_DEV - 1, M_BLK, D_HALF), BF),
            pltpu.VMEM((N_DEV - 1, M_BLK, D_HALF), BF),
            pltpu.VMEM((M_HF, D), jnp.float32),
            pltpu.VMEM((M_HF, D), jnp.float32),
            pltpu.VMEM((3, M_BLK, D), BF),
            pltpu.SemaphoreType.DMA((8,)),
            pltpu.SemaphoreType.DMA((3 + NP,)),
            sem32, sem32, sem32, sem32,
            sem32, sem32, sem32, sem32,
        ],
        compiler_params=pltpu.CompilerParams(collective_id=0),
    )(x, W1b, W2b)
